# baseline (speedup 1.0000x reference)
import functools

import jax
import jax.numpy as jnp
from jax import lax
from jax.experimental import pallas as pl
from jax.experimental.pallas import tpu as pltpu
from jax.experimental.pallas import tpu_sc as plsc

N = 10000
E = 320000
F = 128

NC = 2
NS = 16
NW = NC * NS
EPW = E // NW
CH = 40
NCH = EPW // CH
NB = 8
NR = NCH // NB
TAIL = NCH - NR * NB
NBD = 8

NPAD = 10240
DEG_PT = NPAD // NS
RPT = NPAD // NS

_sc_mesh = plsc.VectorSubcoreMesh(core_axis_name="c", subcore_axis_name="s")


@functools.partial(
    pl.kernel,
    out_type=jax.ShapeDtypeStruct((NC * NPAD,), jnp.float32),
    mesh=_sc_mesh,
    scratch_types=[
        [pltpu.VMEM((CH,), jnp.int32)] * NBD,
        pltpu.VMEM((CH,), jnp.float32),
        pltpu.VMEM((DEG_PT,), jnp.float32),
        pltpu.VMEM_SHARED((NPAD,), jnp.float32),
        pltpu.SemaphoreType.DMA((NBD,)),
        pltpu.SemaphoreType.DMA((NBD,)),
    ],
)
def _deg_kernel(dst_hbm, out_hbm, idxs, ones_v, stage_v, acc_sh, isems, ssems):
    c = lax.axis_index("c")
    s = lax.axis_index("s")
    wid = c * NS + s

    def _init_ones(i, carry):
        ones_v[pl.ds(i * 16, 16)] = jnp.ones((16,), jnp.float32)
        return carry

    lax.fori_loop(0, CH // 16, _init_ones, 0)

    def _init_zero(i, carry):
        stage_v[pl.ds(i * 16, 16)] = jnp.zeros((16,), jnp.float32)
        return carry

    lax.fori_loop(0, DEG_PT // 16, _init_zero, 0)
    pltpu.sync_copy(stage_v, acc_sh.at[pl.ds(s * DEG_PT, DEG_PT)])
    plsc.subcore_barrier()

    def _deg_round(r, nb):
        idn = []
        for j in range(nb):
            k = r * NBD + j
            idn.append(pltpu.async_copy(
                dst_hbm.at[pl.ds(wid * EPW + k * CH, CH)], idxs[j],
                isems.at[j]))
        sd = []
        for j in range(nb):
            idn[j].wait()
            sd.append(pltpu.async_copy(
                ones_v, acc_sh.at[idxs[j]], ssems.at[j], add=True))
        for d in sd:
            d.wait()

    def _round(r, carry):
        _deg_round(r, NBD)
        return carry

    lax.fori_loop(0, NCH // NBD, _round, 0)
    _deg_round(NCH // NBD, NCH - (NCH // NBD) * NBD)
    plsc.subcore_barrier()
    pltpu.sync_copy(acc_sh.at[pl.ds(s * DEG_PT, DEG_PT)], stage_v)
    pltpu.sync_copy(stage_v, out_hbm.at[pl.ds(c * NPAD + s * DEG_PT, DEG_PT)])


@functools.partial(
    pl.kernel,
    out_type=jax.ShapeDtypeStruct((NC * NPAD, F), jnp.float32),
    mesh=_sc_mesh,
    scratch_types=[
        [pltpu.VMEM((CH,), jnp.int32)] * NB,
        [pltpu.VMEM((CH,), jnp.int32)] * NB,
        [pltpu.VMEM((CH, F), jnp.float32)] * NB,
        pltpu.VMEM_SHARED((NPAD, F), jnp.float32),
        pltpu.SemaphoreType.DMA((NB,)),
        pltpu.SemaphoreType.DMA((NB,)),
        pltpu.SemaphoreType.DMA((NB,)),
    ],
)
def _agg_kernel(tab_hbm, src_hbm, dst_hbm, out_hbm,
                sis, dis, rows, acc_sh, isems, gsems, ssems):
    c = lax.axis_index("c")
    s = lax.axis_index("s")
    wid = c * NS + s

    def _zrow(i, carry):
        def _zcol(j, inner):
            rows[0][i, pl.ds(j * 16, 16)] = jnp.zeros((16,), jnp.float32)
            return inner

        return lax.fori_loop(0, F // 16, _zcol, carry)

    lax.fori_loop(0, CH, _zrow, 0)
    for m in range(RPT // CH):
        pltpu.sync_copy(rows[0], acc_sh.at[pl.ds(s * RPT + m * CH, CH)])
    plsc.subcore_barrier()

    def _do_round(r, nb, first, last):
        idn = []
        for j in range(nb):
            if first is None:
                pltpu.make_async_copy(
                    rows[j], acc_sh.at[dis[j]], ssems.at[j]).wait()
            elif first is not True:
                @pl.when(r > 0)
                def _drain(j=j):
                    pltpu.make_async_copy(
                        rows[j], acc_sh.at[dis[j]], ssems.at[j]).wait()
            base = wid * EPW + (r * NB + j) * CH
            idn.append((
                pltpu.async_copy(src_hbm.at[pl.ds(base, CH)], sis[j],
                                 isems.at[j]),
                pltpu.async_copy(dst_hbm.at[pl.ds(base, CH)], dis[j],
                                 gsems.at[j]),
            ))
        gd = []
        for j in range(nb):
            idn[j][0].wait()
            gd.append(pltpu.async_copy(
                tab_hbm.at[sis[j]], rows[j], isems.at[j]))
        sd = []
        for j in range(nb):
            idn[j][1].wait()
            gd[j].wait()
            sd.append(pltpu.async_copy(
                rows[j], acc_sh.at[dis[j]], ssems.at[j], add=True))
        if last:
            for d in sd:
                d.wait()

    def _round(r, carry):
        _do_round(r, NB, first=False, last=False)
        return carry

    lax.fori_loop(0, NR, _round, 0)
    _do_round(NR, TAIL, first=None, last=True)
    for j in range(TAIL, NB):
        pltpu.make_async_copy(rows[j], acc_sh.at[dis[j]], ssems.at[j]).wait()

    plsc.subcore_barrier()
    descs = [None, None]
    for m in range(RPT // CH):
        b = m % 2
        if descs[b] is not None:
            descs[b].wait()
        r0 = s * RPT + m * CH
        pltpu.sync_copy(acc_sh.at[pl.ds(r0, CH)], rows[b])
        descs[b] = pltpu.async_copy(
            rows[b], out_hbm.at[pl.ds(c * NPAD + r0, CH)], gsems.at[b])
    for d in descs:
        if d is not None:
            d.wait()


def _tc1_body(x_ref, normc_ref, w1_ref, o_ref):
    xs = x_ref[...] * normc_ref[...]
    o_ref[...] = jnp.dot(xs, w1_ref[...], preferred_element_type=jnp.float32)


def _tc2_body(pp_ref, h1s_ref, normc_ref, b1_ref, w2_ref, o_ref):
    p01 = pp_ref[...]
    p = p01[:N] + p01[NPAD:NPAD + N]
    z = jnp.maximum((p + h1s_ref[...]) * normc_ref[...] + b1_ref[...], 0.0)
    o_ref[...] = jnp.dot(z * normc_ref[...], w2_ref[...],
                         preferred_element_type=jnp.float32)


def _tc3_body(pp_ref, h2s_ref, normc_ref, b2_ref, wp_ref, bp_ref, o_ref):
    p01 = pp_ref[...]
    p = p01[:N] + p01[NPAD:NPAD + N]
    nr = (p + h2s_ref[...]) * normc_ref[...] + b2_ref[...]
    g = jnp.sum(nr, axis=0, keepdims=True) * (1.0 / N)
    proj = jnp.dot(g, wp_ref[...], preferred_element_type=jnp.float32) \
        + bp_ref[...]
    nrm = jnp.sqrt(jnp.sum(proj * proj, keepdims=True))
    o_ref[...] = proj / jnp.maximum(nrm, 1e-12)


_tc1 = pl.pallas_call(
    _tc1_body, out_shape=jax.ShapeDtypeStruct((N, F), jnp.float32))
_tc2 = pl.pallas_call(
    _tc2_body, out_shape=jax.ShapeDtypeStruct((N, F), jnp.float32))
_tc3 = pl.pallas_call(
    _tc3_body, out_shape=jax.ShapeDtypeStruct((1, F), jnp.float32))


def kernel(x, edge_index, W1, b1, W2, b2, Wp, bp):
    src = edge_index[0]
    dst = edge_index[1]

    degp = _deg_kernel(dst)
    deg = degp[:NPAD] + degp[NPAD:]
    normc = lax.rsqrt(deg[:N] + 1.0).reshape(N, 1)

    h1s = _tc1(x, normc, W1)
    pp1 = _agg_kernel(h1s, src, dst)
    h2s = _tc2(pp1, h1s, normc, b1.reshape(1, F), W2)
    pp2 = _agg_kernel(h2s, src, dst)
    return _tc3(pp2, h2s, normc, b2.reshape(1, F), Wp, bp.reshape(1, F))


# speedup vs baseline: 1.0589x; 1.0589x over previous
import functools

import jax
import jax.numpy as jnp
from jax import lax
from jax.experimental import pallas as pl
from jax.experimental.pallas import tpu as pltpu
from jax.experimental.pallas import tpu_sc as plsc

N = 10000
E = 320000
F = 128

NC = 2
NS = 16
NW = NC * NS
EPW = E // NW
CH = 80
NCH = EPW // CH
NB = 4
NR = NCH // NB
TAIL = NCH - NR * NB
NBD = 8

NPAD = 10240
DEG_PT = NPAD // NS
RPT = NPAD // NS

_sc_mesh = plsc.VectorSubcoreMesh(core_axis_name="c", subcore_axis_name="s")


@functools.partial(
    pl.kernel,
    out_type=jax.ShapeDtypeStruct((NC * NPAD,), jnp.float32),
    mesh=_sc_mesh,
    scratch_types=[
        [pltpu.VMEM((CH,), jnp.int32)] * NBD,
        pltpu.VMEM((CH,), jnp.float32),
        pltpu.VMEM((DEG_PT,), jnp.float32),
        pltpu.VMEM_SHARED((NPAD,), jnp.float32),
        pltpu.SemaphoreType.DMA((NBD,)),
        pltpu.SemaphoreType.DMA((NBD,)),
    ],
)
def _deg_kernel(dst_hbm, out_hbm, idxs, ones_v, stage_v, acc_sh, isems, ssems):
    c = lax.axis_index("c")
    s = lax.axis_index("s")
    wid = c * NS + s

    def _init_ones(i, carry):
        ones_v[pl.ds(i * 16, 16)] = jnp.ones((16,), jnp.float32)
        return carry

    lax.fori_loop(0, CH // 16, _init_ones, 0)

    def _init_zero(i, carry):
        stage_v[pl.ds(i * 16, 16)] = jnp.zeros((16,), jnp.float32)
        return carry

    lax.fori_loop(0, DEG_PT // 16, _init_zero, 0)
    pltpu.sync_copy(stage_v, acc_sh.at[pl.ds(s * DEG_PT, DEG_PT)])
    plsc.subcore_barrier()

    def _deg_round(r, nb):
        idn = []
        for j in range(nb):
            k = r * NBD + j
            idn.append(pltpu.async_copy(
                dst_hbm.at[pl.ds(wid * EPW + k * CH, CH)], idxs[j],
                isems.at[j]))
        sd = []
        for j in range(nb):
            idn[j].wait()
            sd.append(pltpu.async_copy(
                ones_v, acc_sh.at[idxs[j]], ssems.at[j], add=True))
        for d in sd:
            d.wait()

    def _round(r, carry):
        _deg_round(r, NBD)
        return carry

    lax.fori_loop(0, NCH // NBD, _round, 0)
    _deg_round(NCH // NBD, NCH - (NCH // NBD) * NBD)
    plsc.subcore_barrier()
    pltpu.sync_copy(acc_sh.at[pl.ds(s * DEG_PT, DEG_PT)], stage_v)
    pltpu.sync_copy(stage_v, out_hbm.at[pl.ds(c * NPAD + s * DEG_PT, DEG_PT)])


@functools.partial(
    pl.kernel,
    out_type=jax.ShapeDtypeStruct((NC * NPAD, F), jnp.float32),
    mesh=_sc_mesh,
    scratch_types=[
        [pltpu.VMEM((CH,), jnp.int32)] * NB,
        [pltpu.VMEM((CH,), jnp.int32)] * NB,
        [pltpu.VMEM((CH, F), jnp.float32)] * NB,
        pltpu.VMEM_SHARED((NPAD, F), jnp.float32),
        pltpu.SemaphoreType.DMA((NB,)),
        pltpu.SemaphoreType.DMA((NB,)),
        pltpu.SemaphoreType.DMA((NB,)),
    ],
)
def _agg_kernel(tab_hbm, src_hbm, dst_hbm, out_hbm,
                sis, dis, rows, acc_sh, isems, gsems, ssems):
    c = lax.axis_index("c")
    s = lax.axis_index("s")
    wid = c * NS + s

    def _zrow(i, carry):
        def _zcol(j, inner):
            rows[0][i, pl.ds(j * 16, 16)] = jnp.zeros((16,), jnp.float32)
            return inner

        return lax.fori_loop(0, F // 16, _zcol, carry)

    lax.fori_loop(0, CH, _zrow, 0)
    for m in range(RPT // CH):
        pltpu.sync_copy(rows[0], acc_sh.at[pl.ds(s * RPT + m * CH, CH)])
    plsc.subcore_barrier()

    def _do_round(r, nb, first, last):
        idn = []
        for j in range(nb):
            if first is None:
                pltpu.make_async_copy(
                    rows[j], acc_sh.at[dis[j]], ssems.at[j]).wait()
            elif first is not True:
                @pl.when(r > 0)
                def _drain(j=j):
                    pltpu.make_async_copy(
                        rows[j], acc_sh.at[dis[j]], ssems.at[j]).wait()
            base = wid * EPW + (r * NB + j) * CH
            idn.append((
                pltpu.async_copy(src_hbm.at[pl.ds(base, CH)], sis[j],
                                 isems.at[j]),
                pltpu.async_copy(dst_hbm.at[pl.ds(base, CH)], dis[j],
                                 gsems.at[j]),
            ))
        gd = []
        for j in range(nb):
            idn[j][0].wait()
            gd.append(pltpu.async_copy(
                tab_hbm.at[sis[j]], rows[j], isems.at[j]))
        sd = []
        for j in range(nb):
            idn[j][1].wait()
            gd[j].wait()
            sd.append(pltpu.async_copy(
                rows[j], acc_sh.at[dis[j]], ssems.at[j], add=True))
        if last:
            for d in sd:
                d.wait()

    def _round(r, carry):
        _do_round(r, NB, first=False, last=False)
        return carry

    lax.fori_loop(0, NR, _round, 0)
    _do_round(NR, TAIL, first=None, last=True)
    for j in range(TAIL, NB):
        pltpu.make_async_copy(rows[j], acc_sh.at[dis[j]], ssems.at[j]).wait()

    plsc.subcore_barrier()
    descs = [None, None]
    for m in range(RPT // CH):
        b = m % 2
        if descs[b] is not None:
            descs[b].wait()
        r0 = s * RPT + m * CH
        pltpu.sync_copy(acc_sh.at[pl.ds(r0, CH)], rows[b])
        descs[b] = pltpu.async_copy(
            rows[b], out_hbm.at[pl.ds(c * NPAD + r0, CH)], gsems.at[b])
    for d in descs:
        if d is not None:
            d.wait()


NBLK = NPAD // F


def _tc1_body(degp_ref, x_ref, w1_ref, h1s_ref, normc_ref, nscr):
    d2 = degp_ref[...]
    nscr[...] = lax.rsqrt(d2[:NBLK] + d2[NBLK:] + 1.0)

    def _col(a, carry):
        nrow = nscr[pl.ds(a, 1), :]
        normc_ref[pl.ds(a * F, F), :] = jnp.swapaxes(nrow, 0, 1)
        return carry

    lax.fori_loop(0, NBLK, _col, 0)
    h = jnp.dot(x_ref[...], w1_ref[...], preferred_element_type=jnp.float32)
    h1s_ref[...] = h * normc_ref[...]


def _tc2_body(pp_ref, h1s_ref, normc_ref, b1_ref, w2_ref, o_ref):
    p01 = pp_ref[...]
    nc = normc_ref[...]
    p = p01[:NPAD] + p01[NPAD:]
    z = jnp.maximum((p + h1s_ref[...]) * nc + b1_ref[...], 0.0)
    rows = lax.broadcasted_iota(jnp.int32, (NPAD, 1), 0)
    z = jnp.where(rows < N, z, 0.0)
    o_ref[...] = jnp.dot(z * nc, w2_ref[...],
                         preferred_element_type=jnp.float32)


def _tc3_body(pp_ref, h2s_ref, normc_ref, b2_ref, wp_ref, bp_ref, o_ref):
    p01 = pp_ref[...]
    p = p01[:NPAD] + p01[NPAD:]
    nr = (p + h2s_ref[...]) * normc_ref[...]
    g = jnp.sum(nr, axis=0, keepdims=True) * (1.0 / N) + b2_ref[...]
    proj = jnp.dot(g, wp_ref[...], preferred_element_type=jnp.float32) \
        + bp_ref[...]
    nrm = jnp.sqrt(jnp.sum(proj * proj, keepdims=True))
    o_ref[...] = proj / jnp.maximum(nrm, 1e-12)


_tc1 = pl.pallas_call(
    _tc1_body,
    out_shape=[jax.ShapeDtypeStruct((NPAD, F), jnp.float32),
               jax.ShapeDtypeStruct((NPAD, 1), jnp.float32)],
    scratch_shapes=[pltpu.VMEM((NBLK, F), jnp.float32)])
_tc2 = pl.pallas_call(
    _tc2_body, out_shape=jax.ShapeDtypeStruct((NPAD, F), jnp.float32))
_tc3 = pl.pallas_call(
    _tc3_body, out_shape=jax.ShapeDtypeStruct((1, F), jnp.float32))


def kernel(x, edge_index, W1, b1, W2, b2, Wp, bp):
    src = edge_index[0]
    dst = edge_index[1]
    xp = jnp.pad(x, ((0, NPAD - N), (0, 0)))

    degp = _deg_kernel(dst).reshape(2 * NBLK, F)
    h1s, normc = _tc1(degp, xp, W1)
    pp1 = _agg_kernel(h1s, src, dst)
    h2s = _tc2(pp1, h1s, normc, b1.reshape(1, F), W2)
    pp2 = _agg_kernel(h2s, src, dst)
    return _tc3(pp2, h2s, normc, b2.reshape(1, F), Wp, bp.reshape(1, F))


# speedup vs baseline: 1.0758x; 1.0160x over previous
import functools

import jax
import jax.numpy as jnp
from jax import lax
from jax.experimental import pallas as pl
from jax.experimental.pallas import tpu as pltpu
from jax.experimental.pallas import tpu_sc as plsc

N = 10000
E = 320000
F = 128

NC = 2
NS = 16
NW = NC * NS
CH = 128
NCHT = E // CH
CPW = NCHT // NW
XTRA = NCHT - CPW * NW

NB = 3
NR = CPW // NB
NBD = 8

NPAD = 10112
PT = NPAD // NS
NBLK = NPAD // F

_sc_mesh = plsc.VectorSubcoreMesh(core_axis_name="c", subcore_axis_name="s")


@functools.partial(
    pl.kernel,
    out_type=jax.ShapeDtypeStruct((NC * NPAD,), jnp.float32),
    mesh=_sc_mesh,
    scratch_types=[
        [pltpu.VMEM((2, CH), jnp.int32)] * NBD,
        pltpu.VMEM((CH,), jnp.float32),
        pltpu.VMEM((PT,), jnp.float32),
        pltpu.VMEM_SHARED((NPAD,), jnp.float32),
        pltpu.SemaphoreType.DMA((NBD,)),
        pltpu.SemaphoreType.DMA((NBD,)),
    ],
)
def _deg_kernel(ei_hbm, out_hbm, idxs, ones_v, stage_v, acc_sh, isems, ssems):
    c = lax.axis_index("c")
    s = lax.axis_index("s")
    wid = c * NS + s

    def _init_ones(i, carry):
        ones_v[pl.ds(i * 16, 16)] = jnp.ones((16,), jnp.float32)
        return carry

    lax.fori_loop(0, CH // 16, _init_ones, 0)

    def _init_zero(i, carry):
        stage_v[pl.ds(i * 16, 16)] = jnp.zeros((16,), jnp.float32)
        return carry

    lax.fori_loop(0, PT // 16, _init_zero, 0)
    pltpu.sync_copy(stage_v, acc_sh.at[pl.ds(s * PT, PT)])
    plsc.subcore_barrier()

    def _deg_round(r, nb):
        idn = []
        for j in range(nb):
            ck = wid * CPW + r * NBD + j
            idn.append(pltpu.async_copy(
                ei_hbm.at[:, pl.ds(ck * CH, CH)], idxs[j], isems.at[j]))
        sd = []
        for j in range(nb):
            idn[j].wait()
            sd.append(pltpu.async_copy(
                ones_v, acc_sh.at[idxs[j].at[1]], ssems.at[j], add=True))
        for d in sd:
            d.wait()

    def _round(r, carry):
        _deg_round(r, NBD)
        return carry

    lax.fori_loop(0, CPW // NBD, _round, 0)
    _deg_round(CPW // NBD, CPW - (CPW // NBD) * NBD)

    @pl.when(wid < XTRA)
    def _extra():
        ck = NW * CPW + wid
        pltpu.sync_copy(ei_hbm.at[:, pl.ds(ck * CH, CH)], idxs[0])
        pltpu.sync_copy(ones_v, acc_sh.at[idxs[0].at[1]], add=True)

    plsc.subcore_barrier()
    pltpu.sync_copy(acc_sh.at[pl.ds(s * PT, PT)], stage_v)
    pltpu.sync_copy(stage_v, out_hbm.at[pl.ds(c * NPAD + s * PT, PT)])


@functools.partial(
    pl.kernel,
    out_type=jax.ShapeDtypeStruct((NC * NPAD, F), jnp.float32),
    mesh=_sc_mesh,
    scratch_types=[
        [pltpu.VMEM((2, CH), jnp.int32)] * NB,
        [pltpu.VMEM((CH, F), jnp.float32)] * NB,
        pltpu.VMEM_SHARED((NPAD, F), jnp.float32),
        pltpu.SemaphoreType.DMA((NB,)),
        pltpu.SemaphoreType.DMA((NB,)),
        pltpu.SemaphoreType.DMA((NB,)),
    ],
)
def _agg_kernel(tab_hbm, ei_hbm, out_hbm,
                idxs, rows, acc_sh, isems, gsems, ssems):
    c = lax.axis_index("c")
    s = lax.axis_index("s")
    wid = c * NS + s

    def _zrow(i, carry):
        def _zcol(j, inner):
            rows[0][i, pl.ds(j * 16, 16)] = jnp.zeros((16,), jnp.float32)
            return inner

        return lax.fori_loop(0, F // 16, _zcol, carry)

    lax.fori_loop(0, CH, _zrow, 0)
    for m, nr_m in enumerate((128, 128, 128, 128, 120)):
        pltpu.sync_copy(rows[0].at[pl.ds(0, nr_m)],
                        acc_sh.at[pl.ds(s * PT + m * 128, nr_m)])
    plsc.subcore_barrier()

    def _do_round(r, nb, first, last):
        idn = []
        for j in range(nb):
            if first is None:
                pltpu.make_async_copy(
                    rows[j], acc_sh.at[idxs[j].at[1]], ssems.at[j]).wait()
            elif not first:
                @pl.when(r > 0)
                def _drain(j=j):
                    pltpu.make_async_copy(
                        rows[j], acc_sh.at[idxs[j].at[1]], ssems.at[j]).wait()
            ck = wid * CPW + r * NB + j
            idn.append(pltpu.async_copy(
                ei_hbm.at[:, pl.ds(ck * CH, CH)], idxs[j], isems.at[j]))
        gd = []
        for j in range(nb):
            idn[j].wait()
            gd.append(pltpu.async_copy(
                tab_hbm.at[idxs[j].at[0]], rows[j], gsems.at[j]))
        sd = []
        for j in range(nb):
            gd[j].wait()
            sd.append(pltpu.async_copy(
                rows[j], acc_sh.at[idxs[j].at[1]], ssems.at[j], add=True))
        if last:
            for d in sd:
                d.wait()

    def _round(r, carry):
        _do_round(r, NB, first=False, last=False)
        return carry

    _do_round(0, NB, first=True, last=False)
    lax.fori_loop(1, NR, _round, 0)
    for j in range(NB):
        pltpu.make_async_copy(
            rows[j], acc_sh.at[idxs[j].at[1]], ssems.at[j]).wait()

    @pl.when(wid < XTRA)
    def _extra():
        ck = NW * CPW + wid
        pltpu.sync_copy(ei_hbm.at[:, pl.ds(ck * CH, CH)], idxs[0])
        pltpu.sync_copy(tab_hbm.at[idxs[0].at[0]], rows[0])
        pltpu.sync_copy(rows[0], acc_sh.at[idxs[0].at[1]], add=True)

    plsc.subcore_barrier()
    descs = [None, None]
    for m, nr_m in enumerate((128, 128, 128, 128, 120)):
        b = m % 2
        if descs[b] is not None:
            descs[b].wait()
        r0 = s * PT + m * 128
        pltpu.sync_copy(acc_sh.at[pl.ds(r0, nr_m)], rows[b].at[pl.ds(0, nr_m)])
        descs[b] = pltpu.async_copy(
            rows[b].at[pl.ds(0, nr_m)], out_hbm.at[pl.ds(c * NPAD + r0, nr_m)],
            gsems.at[b])
    for d in descs:
        if d is not None:
            d.wait()


def _tcmm_body(x_ref, w1_ref, o_ref):
    h = jnp.dot(x_ref[...], w1_ref[...], preferred_element_type=jnp.float32)
    o_ref[...] = jnp.concatenate(
        [h, jnp.zeros((NPAD - N, F), jnp.float32)], axis=0)


def _tc1b_body(degp_ref, h1_ref, h1s_ref, normc_ref, nscr):
    d2 = degp_ref[...]
    nscr[...] = lax.rsqrt(d2[:NBLK] + d2[NBLK:] + 1.0)

    def _col(a, carry):
        nrow = nscr[pl.ds(a, 1), :]
        normc_ref[pl.ds(a * F, F), :] = jnp.swapaxes(nrow, 0, 1)
        return carry

    lax.fori_loop(0, NBLK, _col, 0)
    h1s_ref[...] = h1_ref[...] * normc_ref[...]


def _tc2_body(pp_ref, h1s_ref, normc_ref, b1_ref, w2_ref, o_ref):
    p01 = pp_ref[...]
    nc = normc_ref[...]
    p = p01[:NPAD] + p01[NPAD:]
    z = jnp.maximum((p + h1s_ref[...]) * nc + b1_ref[...], 0.0)
    rows = lax.broadcasted_iota(jnp.int32, (NPAD, 1), 0)
    z = jnp.where(rows < N, z, 0.0)
    o_ref[...] = jnp.dot(z * nc, w2_ref[...],
                         preferred_element_type=jnp.float32)


def _tc3_body(pp_ref, h2s_ref, normc_ref, b2_ref, wp_ref, bp_ref, o_ref):
    p01 = pp_ref[...]
    p = p01[:NPAD] + p01[NPAD:]
    nr = (p + h2s_ref[...]) * normc_ref[...]
    g = jnp.sum(nr, axis=0, keepdims=True) * (1.0 / N) + b2_ref[...]
    proj = jnp.dot(g, wp_ref[...], preferred_element_type=jnp.float32) \
        + bp_ref[...]
    nrm = jnp.sqrt(jnp.sum(proj * proj, keepdims=True))
    o_ref[...] = proj / jnp.maximum(nrm, 1e-12)


_tcmm = pl.pallas_call(
    _tcmm_body, out_shape=jax.ShapeDtypeStruct((NPAD, F), jnp.float32))
_tc1b = pl.pallas_call(
    _tc1b_body,
    out_shape=[jax.ShapeDtypeStruct((NPAD, F), jnp.float32),
               jax.ShapeDtypeStruct((NPAD, 1), jnp.float32)],
    scratch_shapes=[pltpu.VMEM((NBLK, F), jnp.float32)])
_tc2 = pl.pallas_call(
    _tc2_body, out_shape=jax.ShapeDtypeStruct((NPAD, F), jnp.float32))
_tc3 = pl.pallas_call(
    _tc3_body, out_shape=jax.ShapeDtypeStruct((1, F), jnp.float32))


def kernel(x, edge_index, W1, b1, W2, b2, Wp, bp):
    degp = _deg_kernel(edge_index).reshape(2 * NBLK, F)
    h1 = _tcmm(x, W1)
    h1s, normc = _tc1b(degp, h1)
    pp1 = _agg_kernel(h1s, edge_index)
    h2s = _tc2(pp1, h1s, normc, b1.reshape(1, F), W2)
    pp2 = _agg_kernel(h2s, edge_index)
    return _tc3(pp2, h2s, normc, b2.reshape(1, F), Wp, bp.reshape(1, F))


# speedup vs baseline: 1.0793x; 1.0033x over previous
import functools

import jax
import jax.numpy as jnp
from jax import lax
from jax.experimental import pallas as pl
from jax.experimental.pallas import tpu as pltpu
from jax.experimental.pallas import tpu_sc as plsc

N = 10000
E = 320000
F = 128

NC = 2
NS = 16
NW = NC * NS
CH = 128
NCHT = E // CH
CPW = NCHT // NW
XTRA = NCHT - CPW * NW

NB = 3
NR = CPW // NB
NBD = 8

NPAD = 10112
PT = NPAD // NS
NBLK = NPAD // F

_sc_mesh = plsc.VectorSubcoreMesh(core_axis_name="c", subcore_axis_name="s")


@functools.partial(
    pl.kernel,
    out_type=jax.ShapeDtypeStruct((NC * NPAD,), jnp.float32),
    mesh=_sc_mesh,
    scratch_types=[
        [pltpu.VMEM((2, CH), jnp.int32)] * NBD,
        pltpu.VMEM((CH,), jnp.float32),
        pltpu.VMEM((PT,), jnp.float32),
        pltpu.VMEM_SHARED((NPAD,), jnp.float32),
        pltpu.SemaphoreType.DMA((NBD,)),
        pltpu.SemaphoreType.DMA((NBD,)),
    ],
)
def _deg_kernel(ei_hbm, out_hbm, idxs, ones_v, stage_v, acc_sh, isems, ssems):
    c = lax.axis_index("c")
    s = lax.axis_index("s")
    wid = c * NS + s

    def _init_ones(i, carry):
        ones_v[pl.ds(i * 16, 16)] = jnp.ones((16,), jnp.float32)
        return carry

    lax.fori_loop(0, CH // 16, _init_ones, 0)

    def _init_zero(i, carry):
        stage_v[pl.ds(i * 16, 16)] = jnp.zeros((16,), jnp.float32)
        return carry

    lax.fori_loop(0, PT // 16, _init_zero, 0)
    pltpu.sync_copy(stage_v, acc_sh.at[pl.ds(s * PT, PT)])
    plsc.subcore_barrier()

    def _deg_round(r, nb):
        idn = []
        for j in range(nb):
            ck = wid * CPW + r * NBD + j
            idn.append(pltpu.async_copy(
                ei_hbm.at[:, pl.ds(ck * CH, CH)], idxs[j], isems.at[j]))
        sd = []
        for j in range(nb):
            idn[j].wait()
            sd.append(pltpu.async_copy(
                ones_v, acc_sh.at[idxs[j].at[1]], ssems.at[j], add=True))
        for d in sd:
            d.wait()

    def _round(r, carry):
        _deg_round(r, NBD)
        return carry

    lax.fori_loop(0, CPW // NBD, _round, 0)
    _deg_round(CPW // NBD, CPW - (CPW // NBD) * NBD)

    @pl.when(wid < XTRA)
    def _extra():
        ck = NW * CPW + wid
        pltpu.sync_copy(ei_hbm.at[:, pl.ds(ck * CH, CH)], idxs[0])
        pltpu.sync_copy(ones_v, acc_sh.at[idxs[0].at[1]], add=True)

    plsc.subcore_barrier()
    pltpu.sync_copy(acc_sh.at[pl.ds(s * PT, PT)], stage_v)
    pltpu.sync_copy(stage_v, out_hbm.at[pl.ds(c * NPAD + s * PT, PT)])


@functools.partial(
    pl.kernel,
    out_type=jax.ShapeDtypeStruct((NC * NPAD, F), jnp.float32),
    mesh=_sc_mesh,
    scratch_types=[
        [pltpu.VMEM((2, CH), jnp.int32)] * NB,
        [pltpu.VMEM((CH, F), jnp.float32)] * NB,
        pltpu.VMEM_SHARED((NPAD, F), jnp.float32),
        pltpu.SemaphoreType.DMA((NB,)),
        pltpu.SemaphoreType.DMA((NB,)),
        pltpu.SemaphoreType.DMA((NB,)),
    ],
)
def _agg_kernel(tab_hbm, ei_hbm, out_hbm,
                idxs, rows, acc_sh, isems, gsems, ssems):
    c = lax.axis_index("c")
    s = lax.axis_index("s")
    wid = c * NS + s

    def _zrow(i, carry):
        def _zcol(j, inner):
            rows[0][i, pl.ds(j * 16, 16)] = jnp.zeros((16,), jnp.float32)
            return inner

        return lax.fori_loop(0, F // 16, _zcol, carry)

    lax.fori_loop(0, CH, _zrow, 0)
    for m, nr_m in enumerate((128, 128, 128, 128, 120)):
        pltpu.sync_copy(rows[0].at[pl.ds(0, nr_m)],
                        acc_sh.at[pl.ds(s * PT + m * 128, nr_m)])
    plsc.subcore_barrier()

    def _do_round(r, nb, first, last):
        idn = []
        for j in range(nb):
            if first is None:
                pltpu.make_async_copy(
                    rows[j], acc_sh.at[idxs[j].at[1]], ssems.at[j]).wait()
            elif not first:
                @pl.when(r > 0)
                def _drain(j=j):
                    pltpu.make_async_copy(
                        rows[j], acc_sh.at[idxs[j].at[1]], ssems.at[j]).wait()
            ck = wid * CPW + r * NB + j
            idn.append(pltpu.async_copy(
                ei_hbm.at[:, pl.ds(ck * CH, CH)], idxs[j], isems.at[j]))
        gd = []
        for j in range(nb):
            idn[j].wait()
            gd.append((
                pltpu.async_copy(
                    tab_hbm.at[idxs[j].at[0, pl.ds(0, CH // 2)]],
                    rows[j].at[pl.ds(0, CH // 2)], gsems.at[j]),
                pltpu.async_copy(
                    tab_hbm.at[idxs[j].at[0, pl.ds(CH // 2, CH // 2)]],
                    rows[j].at[pl.ds(CH // 2, CH // 2)], gsems.at[j]),
            ))
        sd = []
        for j in range(nb):
            gd[j][0].wait()
            gd[j][1].wait()
            sd.append(pltpu.async_copy(
                rows[j], acc_sh.at[idxs[j].at[1]], ssems.at[j], add=True))
        if last:
            for d in sd:
                d.wait()

    def _round(r, carry):
        _do_round(r, NB, first=False, last=False)
        return carry

    _do_round(0, NB, first=True, last=False)
    lax.fori_loop(1, NR, _round, 0)
    for j in range(NB):
        pltpu.make_async_copy(
            rows[j], acc_sh.at[idxs[j].at[1]], ssems.at[j]).wait()

    @pl.when(wid < XTRA)
    def _extra():
        ck = NW * CPW + wid
        pltpu.sync_copy(ei_hbm.at[:, pl.ds(ck * CH, CH)], idxs[0])
        pltpu.sync_copy(tab_hbm.at[idxs[0].at[0]], rows[0])
        pltpu.sync_copy(rows[0], acc_sh.at[idxs[0].at[1]], add=True)

    plsc.subcore_barrier()
    descs = [None, None]
    for m, nr_m in enumerate((128, 128, 128, 128, 120)):
        b = m % 2
        if descs[b] is not None:
            descs[b].wait()
        r0 = s * PT + m * 128
        pltpu.sync_copy(acc_sh.at[pl.ds(r0, nr_m)], rows[b].at[pl.ds(0, nr_m)])
        descs[b] = pltpu.async_copy(
            rows[b].at[pl.ds(0, nr_m)], out_hbm.at[pl.ds(c * NPAD + r0, nr_m)],
            gsems.at[b])
    for d in descs:
        if d is not None:
            d.wait()


def _tcmm_body(x_ref, w1_ref, o_ref):
    h = jnp.dot(x_ref[...], w1_ref[...], preferred_element_type=jnp.float32)
    o_ref[...] = jnp.concatenate(
        [h, jnp.zeros((NPAD - N, F), jnp.float32)], axis=0)


def _tc1b_body(degp_ref, h1_ref, h1s_ref, normc_ref, nscr):
    d2 = degp_ref[...]
    nscr[...] = lax.rsqrt(d2[:NBLK] + d2[NBLK:] + 1.0)

    def _col(a, carry):
        nrow = nscr[pl.ds(a, 1), :]
        normc_ref[pl.ds(a * F, F), :] = jnp.swapaxes(nrow, 0, 1)
        return carry

    lax.fori_loop(0, NBLK, _col, 0)
    h1s_ref[...] = h1_ref[...] * normc_ref[...]


def _tc2_body(pp_ref, h1s_ref, normc_ref, b1_ref, w2_ref, o_ref):
    p01 = pp_ref[...]
    nc = normc_ref[...]
    p = p01[:NPAD] + p01[NPAD:]
    z = jnp.maximum((p + h1s_ref[...]) * nc + b1_ref[...], 0.0)
    rows = lax.broadcasted_iota(jnp.int32, (NPAD, 1), 0)
    z = jnp.where(rows < N, z, 0.0)
    o_ref[...] = jnp.dot(z * nc, w2_ref[...],
                         preferred_element_type=jnp.float32)


def _tc3_body(pp_ref, h2s_ref, normc_ref, b2_ref, wp_ref, bp_ref, o_ref):
    p01 = pp_ref[...]
    p = p01[:NPAD] + p01[NPAD:]
    nr = (p + h2s_ref[...]) * normc_ref[...]
    g = jnp.sum(nr, axis=0, keepdims=True) * (1.0 / N) + b2_ref[...]
    proj = jnp.dot(g, wp_ref[...], preferred_element_type=jnp.float32) \
        + bp_ref[...]
    nrm = jnp.sqrt(jnp.sum(proj * proj, keepdims=True))
    o_ref[...] = proj / jnp.maximum(nrm, 1e-12)


_tcmm = pl.pallas_call(
    _tcmm_body, out_shape=jax.ShapeDtypeStruct((NPAD, F), jnp.float32))
_tc1b = pl.pallas_call(
    _tc1b_body,
    out_shape=[jax.ShapeDtypeStruct((NPAD, F), jnp.float32),
               jax.ShapeDtypeStruct((NPAD, 1), jnp.float32)],
    scratch_shapes=[pltpu.VMEM((NBLK, F), jnp.float32)])
_tc2 = pl.pallas_call(
    _tc2_body, out_shape=jax.ShapeDtypeStruct((NPAD, F), jnp.float32))
_tc3 = pl.pallas_call(
    _tc3_body, out_shape=jax.ShapeDtypeStruct((1, F), jnp.float32))


def kernel(x, edge_index, W1, b1, W2, b2, Wp, bp):
    degp = _deg_kernel(edge_index).reshape(2 * NBLK, F)
    h1 = _tcmm(x, W1)
    h1s, normc = _tc1b(degp, h1)
    pp1 = _agg_kernel(h1s, edge_index)
    h2s = _tc2(pp1, h1s, normc, b1.reshape(1, F), W2)
    pp2 = _agg_kernel(h2s, edge_index)
    return _tc3(pp2, h2s, normc, b2.reshape(1, F), Wp, bp.reshape(1, F))


# speedup vs baseline: 1.0966x; 1.0161x over previous
import functools

import jax
import jax.numpy as jnp
from jax import lax
from jax.experimental import pallas as pl
from jax.experimental.pallas import tpu as pltpu
from jax.experimental.pallas import tpu_sc as plsc

N = 10000
E = 320000
F = 128

NC = 2
NS = 16
NW = NC * NS
CH = 128
NCHT = E // CH
CPW = NCHT // NW
XTRA = NCHT - CPW * NW

NB = 3
NR = CPW // NB
NBD = 8

NPAD = 10112
PT = NPAD // NS
NBLK = NPAD // F

_sc_mesh = plsc.VectorSubcoreMesh(core_axis_name="c", subcore_axis_name="s")


@functools.partial(
    pl.kernel,
    out_type=jax.ShapeDtypeStruct((NC * NPAD,), jnp.float32),
    mesh=_sc_mesh,
    scratch_types=[
        [pltpu.VMEM((2, CH), jnp.int32)] * NBD,
        pltpu.VMEM((CH,), jnp.float32),
        pltpu.VMEM((PT,), jnp.float32),
        pltpu.VMEM_SHARED((NPAD,), jnp.float32),
        pltpu.SemaphoreType.DMA((NBD,)),
        pltpu.SemaphoreType.DMA((NBD,)),
    ],
)
def _deg_kernel(ei_hbm, out_hbm, idxs, ones_v, stage_v, acc_sh, isems, ssems):
    c = lax.axis_index("c")
    s = lax.axis_index("s")
    wid = c * NS + s

    def _init_ones(i, carry):
        ones_v[pl.ds(i * 16, 16)] = jnp.ones((16,), jnp.float32)
        return carry

    lax.fori_loop(0, CH // 16, _init_ones, 0)

    def _init_zero(i, carry):
        stage_v[pl.ds(i * 16, 16)] = jnp.zeros((16,), jnp.float32)
        return carry

    lax.fori_loop(0, PT // 16, _init_zero, 0)
    pltpu.sync_copy(stage_v, acc_sh.at[pl.ds(s * PT, PT)])
    plsc.subcore_barrier()

    def _deg_round(r, nb):
        idn = []
        for j in range(nb):
            ck = wid * CPW + r * NBD + j
            idn.append(pltpu.async_copy(
                ei_hbm.at[:, pl.ds(ck * CH, CH)], idxs[j], isems.at[j]))
        sd = []
        for j in range(nb):
            idn[j].wait()
            sd.append(pltpu.async_copy(
                ones_v, acc_sh.at[idxs[j].at[1]], ssems.at[j], add=True))
        for d in sd:
            d.wait()

    def _round(r, carry):
        _deg_round(r, NBD)
        return carry

    lax.fori_loop(0, CPW // NBD, _round, 0)
    _deg_round(CPW // NBD, CPW - (CPW // NBD) * NBD)

    @pl.when(wid < XTRA)
    def _extra():
        ck = NW * CPW + wid
        pltpu.sync_copy(ei_hbm.at[:, pl.ds(ck * CH, CH)], idxs[0])
        pltpu.sync_copy(ones_v, acc_sh.at[idxs[0].at[1]], add=True)

    plsc.subcore_barrier()
    pltpu.sync_copy(acc_sh.at[pl.ds(s * PT, PT)], stage_v)
    pltpu.sync_copy(stage_v, out_hbm.at[pl.ds(c * NPAD + s * PT, PT)])


@functools.partial(
    pl.kernel,
    out_type=jax.ShapeDtypeStruct((NC * NPAD, F), jnp.float32),
    mesh=_sc_mesh,
    scratch_types=[
        [pltpu.VMEM((2, CH), jnp.int32)] * NB,
        [pltpu.VMEM((CH, F), jnp.float32)] * NB,
        pltpu.VMEM_SHARED((NPAD, F), jnp.float32),
        pltpu.SemaphoreType.DMA((NB,)),
        pltpu.SemaphoreType.DMA((NB,)),
        pltpu.SemaphoreType.DMA((NB,)),
    ],
)
def _agg_kernel(tab_hbm, ei_hbm, out_hbm,
                idxs, rows, acc_sh, isems, gsems, ssems):
    c = lax.axis_index("c")
    s = lax.axis_index("s")
    wid = c * NS + s

    def _zrow(i, carry):
        def _zcol(j, inner):
            rows[0][i, pl.ds(j * 16, 16)] = jnp.zeros((16,), jnp.float32)
            return inner

        return lax.fori_loop(0, F // 16, _zcol, carry)

    lax.fori_loop(0, CH, _zrow, 0)
    for m, nr_m in enumerate((128, 128, 128, 128, 120)):
        pltpu.sync_copy(rows[0].at[pl.ds(0, nr_m)],
                        acc_sh.at[pl.ds(s * PT + m * 128, nr_m)])
    plsc.subcore_barrier()

    def _do_round(r, nb, first, last):
        idn = []
        for j in range(nb):
            if first is None:
                pltpu.make_async_copy(
                    rows[j], acc_sh.at[idxs[j].at[1]], ssems.at[j]).wait()
            elif not first:
                @pl.when(r > 0)
                def _drain(j=j):
                    pltpu.make_async_copy(
                        rows[j], acc_sh.at[idxs[j].at[1]], ssems.at[j]).wait()
            ck = wid * CPW + r * NB + j
            idn.append(pltpu.async_copy(
                ei_hbm.at[:, pl.ds(ck * CH, CH)], idxs[j], isems.at[j]))
        gd = []
        for j in range(nb):
            idn[j].wait()
            gd.append(pltpu.async_copy(
                tab_hbm.at[idxs[j].at[0]], rows[j], gsems.at[j]))
        sd = []
        for j in range(nb):
            gd[j].wait()
            sd.append(pltpu.async_copy(
                rows[j], acc_sh.at[idxs[j].at[1]], ssems.at[j], add=True))
        if last:
            for d in sd:
                d.wait()

    def _round(r, carry):
        _do_round(r, NB, first=False, last=False)
        return carry

    _do_round(0, NB, first=True, last=False)
    lax.fori_loop(1, NR, _round, 0)
    for j in range(NB):
        pltpu.make_async_copy(
            rows[j], acc_sh.at[idxs[j].at[1]], ssems.at[j]).wait()

    @pl.when(wid < XTRA)
    def _extra():
        ck = NW * CPW + wid
        pltpu.sync_copy(ei_hbm.at[:, pl.ds(ck * CH, CH)], idxs[0])
        pltpu.sync_copy(tab_hbm.at[idxs[0].at[0]], rows[0])
        pltpu.sync_copy(rows[0], acc_sh.at[idxs[0].at[1]], add=True)

    plsc.subcore_barrier()
    descs = [None, None]
    for m, nr_m in enumerate((128, 128, 128, 128, 120)):
        b = m % 2
        if descs[b] is not None:
            descs[b].wait()
        r0 = s * PT + m * 128
        pltpu.sync_copy(acc_sh.at[pl.ds(r0, nr_m)], rows[b].at[pl.ds(0, nr_m)])
        descs[b] = pltpu.async_copy(
            rows[b].at[pl.ds(0, nr_m)], out_hbm.at[pl.ds(c * NPAD + r0, nr_m)],
            gsems.at[b])
    for d in descs:
        if d is not None:
            d.wait()


def _tcmm_body(x_ref, w1_ref, o_ref):
    h = jnp.dot(x_ref[...], w1_ref[...], preferred_element_type=jnp.float32)
    o_ref[...] = jnp.concatenate(
        [h, jnp.zeros((NPAD - N, F), jnp.float32)], axis=0)


def _tc1b_body(degp_ref, h1_ref, h1s_ref, normc_ref, nscr):
    d2 = degp_ref[...]
    n80 = lax.rsqrt(d2[:NBLK] + d2[NBLK:] + 1.0)
    rows = lax.broadcasted_iota(jnp.int32, (NPAD, 1), 0)
    blk = lax.broadcasted_iota(jnp.int32, (NPAD, NBLK), 1)
    sel = jnp.where(blk == rows // F, 1.0, 0.0)
    rep = jnp.dot(sel, n80, preferred_element_type=jnp.float32)
    lane = lax.broadcasted_iota(jnp.int32, (NPAD, F), 1)
    nc = jnp.sum(jnp.where(lane == rows % F, rep, 0.0), axis=1, keepdims=True)
    normc_ref[...] = nc
    h1s_ref[...] = h1_ref[...] * nc
    del nscr


def _tc2_body(pp_ref, h1s_ref, normc_ref, b1_ref, w2_ref, o_ref):
    p01 = pp_ref[...]
    nc = normc_ref[...]
    p = p01[:NPAD] + p01[NPAD:]
    z = jnp.maximum((p + h1s_ref[...]) * nc + b1_ref[...], 0.0)
    rows = lax.broadcasted_iota(jnp.int32, (NPAD, 1), 0)
    z = jnp.where(rows < N, z, 0.0)
    o_ref[...] = jnp.dot(z * nc, w2_ref[...],
                         preferred_element_type=jnp.float32)


def _tc3_body(pp_ref, h2s_ref, normc_ref, b2_ref, wp_ref, bp_ref, o_ref):
    p01 = pp_ref[...]
    p = p01[:NPAD] + p01[NPAD:]
    nr = (p + h2s_ref[...]) * normc_ref[...]
    g = jnp.sum(nr, axis=0, keepdims=True) * (1.0 / N) + b2_ref[...]
    proj = jnp.dot(g, wp_ref[...], preferred_element_type=jnp.float32) \
        + bp_ref[...]
    nrm = jnp.sqrt(jnp.sum(proj * proj, keepdims=True))
    o_ref[...] = proj / jnp.maximum(nrm, 1e-12)


_tcmm = pl.pallas_call(
    _tcmm_body, out_shape=jax.ShapeDtypeStruct((NPAD, F), jnp.float32))
_tc1b = pl.pallas_call(
    _tc1b_body,
    out_shape=[jax.ShapeDtypeStruct((NPAD, F), jnp.float32),
               jax.ShapeDtypeStruct((NPAD, 1), jnp.float32)],
    scratch_shapes=[pltpu.VMEM((NBLK, F), jnp.float32)])
_tc2 = pl.pallas_call(
    _tc2_body, out_shape=jax.ShapeDtypeStruct((NPAD, F), jnp.float32))
_tc3 = pl.pallas_call(
    _tc3_body, out_shape=jax.ShapeDtypeStruct((1, F), jnp.float32))


def kernel(x, edge_index, W1, b1, W2, b2, Wp, bp):
    degp = _deg_kernel(edge_index).reshape(2 * NBLK, F)
    h1 = _tcmm(x, W1)
    h1s, normc = _tc1b(degp, h1)
    pp1 = _agg_kernel(h1s, edge_index)
    h2s = _tc2(pp1, h1s, normc, b1.reshape(1, F), W2)
    pp2 = _agg_kernel(h2s, edge_index)
    return _tc3(pp2, h2s, normc, b2.reshape(1, F), Wp, bp.reshape(1, F))
